# Initial kernel scaffold; baseline (speedup 1.0000x reference)
import jax
import jax.numpy as jnp
from jax.experimental import pallas as pl


def kernel(shapes, colors, clusters, mask, shape_table, color_table, cluster_table, W1, b1, W2, b2):
    raise NotImplementedError("write your pallas kernel here")


# speedup vs baseline: 1.0831x; 1.0831x over previous
import functools

import jax
import jax.numpy as jnp
from jax import lax
from jax.experimental import pallas as pl
from jax.experimental.pallas import tpu as pltpu
from jax.experimental.pallas import tpu_sc as plsc

B = 4096
L = 200
EMB = 16
HID = 64
NCLS = 100

NC = 2
NS = 16
NW = NC * NS
BPW = B // NW

CHUNKS = ((0, 104), (104, 96))


def _make_pool_kernel():
    mesh = plsc.VectorSubcoreMesh(core_axis_name="c", subcore_axis_name="s")

    @functools.partial(
        pl.kernel,
        mesh=mesh,
        out_type=jax.ShapeDtypeStruct((3, B, EMB), jnp.float32),
        compiler_params=pltpu.CompilerParams(use_tc_tiling_on_sc=False),
        scratch_types=[
            pltpu.VMEM((BPW * L,), jnp.int32),
            pltpu.VMEM((L, EMB), jnp.float32),
            pltpu.VMEM((BPW, EMB), jnp.float32),
            pltpu.SemaphoreType.DMA,
        ],
    )
    def pool(shapes_hbm, colors_hbm, clusters_hbm, t0, t1, t2, out_hbm,
             idx_v, rows_v, acc_v, sem):
        wid = lax.axis_index("s") * NC + lax.axis_index("c")
        base = wid * BPW
        for t, (idx_hbm, tab) in enumerate(
                ((shapes_hbm, t0), (colors_hbm, t1), (clusters_hbm, t2))):
            pltpu.sync_copy(idx_hbm.at[pl.ds(base * L, BPW * L)], idx_v)

            def row_body(i, carry):
                cps = [
                    pltpu.async_copy(
                        tab.at[idx_v.at[pl.ds(
                            pl.multiple_of(i * L + off, 8), n)]],
                        rows_v.at[pl.ds(off, n)], sem)
                    for off, n in CHUNKS
                ]
                for cp in cps:
                    cp.wait()

                def add_body(j, acc):
                    return acc + rows_v[j]

                acc = lax.fori_loop(0, L, add_body,
                                    jnp.zeros((EMB,), jnp.float32))
                acc_v[i] = acc
                return carry

            lax.fori_loop(0, BPW, row_body, 0)
            pltpu.sync_copy(acc_v, out_hbm.at[t].at[pl.ds(base, BPW)])

    return pool


_pool = _make_pool_kernel()


def _mlp_body(p0, p1, p2, m, w1, b1, w2, b2, o):
    s = jnp.dot(p0[0], w1[0], preferred_element_type=jnp.float32)
    s = s + jnp.dot(p1[0], w1[1], preferred_element_type=jnp.float32)
    s = s + jnp.dot(p2[0], w1[2], preferred_element_type=jnp.float32)
    msum = jnp.sum(m[...], axis=1, keepdims=True)
    h = jnp.maximum(s / msum + b1[...], 0.0)
    o[...] = jnp.dot(h, w2[...], preferred_element_type=jnp.float32) + b2[...]


def _mlp(psum3, mask, W1r, b1r, W2, b2r):
    BB = 512
    grid = (B // BB,)
    return pl.pallas_call(
        _mlp_body,
        grid=grid,
        in_specs=[
            pl.BlockSpec((1, BB, EMB), lambda b: (0, b, 0)),
            pl.BlockSpec((1, BB, EMB), lambda b: (1, b, 0)),
            pl.BlockSpec((1, BB, EMB), lambda b: (2, b, 0)),
            pl.BlockSpec((BB, L), lambda b: (b, 0)),
            pl.BlockSpec((3, EMB, HID), lambda b: (0, 0, 0)),
            pl.BlockSpec((1, HID), lambda b: (0, 0)),
            pl.BlockSpec((HID, NCLS), lambda b: (0, 0)),
            pl.BlockSpec((1, NCLS), lambda b: (0, 0)),
        ],
        out_specs=pl.BlockSpec((BB, NCLS), lambda b: (b, 0)),
        out_shape=jax.ShapeDtypeStruct((B, NCLS), jnp.float32),
    )(psum3, psum3, psum3, mask, W1r, b1r, W2, b2r)


def kernel(shapes, colors, clusters, mask, shape_table, color_table,
           cluster_table, W1, b1, W2, b2):
    psum3 = _pool(shapes.reshape(-1), colors.reshape(-1),
                  clusters.reshape(-1),
                  shape_table, color_table, cluster_table)
    return _mlp(psum3, mask, W1.reshape(3, EMB, HID),
                b1.reshape(1, HID), W2, b2.reshape(1, NCLS))


# speedup vs baseline: 1.5226x; 1.4058x over previous
import functools

import jax
import jax.numpy as jnp
from jax import lax
from jax.experimental import pallas as pl
from jax.experimental.pallas import tpu as pltpu
from jax.experimental.pallas import tpu_sc as plsc

B = 4096
L = 200
EMB = 16
HID = 64
NCLS = 100

NC = 2
NS = 16
NW = NC * NS
BPW = B // NW

R = 4
GROUP = R * L
NG = BPW // R
GCHUNKS = tuple((off, min(128, GROUP - off)) for off in range(0, GROUP, 128))


def _make_pool_kernel():
    mesh = plsc.VectorSubcoreMesh(core_axis_name="c", subcore_axis_name="s")

    @functools.partial(
        pl.kernel,
        mesh=mesh,
        out_type=jax.ShapeDtypeStruct((3, B, EMB), jnp.float32),
        compiler_params=pltpu.CompilerParams(use_tc_tiling_on_sc=False),
        scratch_types=[
            pltpu.VMEM((BPW * L,), jnp.int32),
            pltpu.VMEM((GROUP, EMB), jnp.float32),
            pltpu.VMEM((GROUP, EMB), jnp.float32),
            pltpu.VMEM((BPW, EMB), jnp.float32),
            pltpu.SemaphoreType.DMA,
            pltpu.SemaphoreType.DMA,
        ],
    )
    def pool(shapes_hbm, colors_hbm, clusters_hbm, t0, t1, t2, out_hbm,
             idx_v, buf_a, buf_b, acc_v, sem_a, sem_b):
        wid = lax.axis_index("s") * NC + lax.axis_index("c")
        base = wid * BPW
        zero = jnp.zeros((EMB,), jnp.float32)

        for t, (idx_hbm, tab) in enumerate(
                ((shapes_hbm, t0), (colors_hbm, t1), (clusters_hbm, t2))):
            pltpu.sync_copy(idx_hbm.at[pl.ds(base * L, BPW * L)], idx_v)

            def issue(g, buf, sem):
                gbase = g * GROUP
                for off, n in GCHUNKS:
                    pltpu.async_copy(
                        tab.at[idx_v.at[pl.ds(
                            pl.multiple_of(gbase + off, 8), n)]],
                        buf.at[pl.ds(off, n)], sem)

            def wait(buf, sem):
                pltpu.make_async_copy(
                    out_hbm.at[0].at[pl.ds(0, GROUP)], buf, sem).wait()

            def accum(g, buf):
                def row_body(r, carry):
                    def elem_body(j, accs):
                        a0, a1, a2, a3 = accs
                        b = r * L + j * 20
                        for u in range(0, 20, 4):
                            a0 = a0 + buf[b + u]
                            a1 = a1 + buf[b + u + 1]
                            a2 = a2 + buf[b + u + 2]
                            a3 = a3 + buf[b + u + 3]
                        return a0, a1, a2, a3
                    accs = lax.fori_loop(0, L // 20, elem_body,
                                         (zero, zero, zero, zero))
                    acc_v[g * R + r] = (accs[0] + accs[1]) + (accs[2] + accs[3])
                    return carry
                lax.fori_loop(0, R, row_body, 0)

            issue(0, buf_a, sem_a)
            issue(1, buf_b, sem_b)

            def pair_body(k, carry):
                wait(buf_a, sem_a)
                accum(2 * k, buf_a)
                issue(2 * k + 2, buf_a, sem_a)
                wait(buf_b, sem_b)
                accum(2 * k + 1, buf_b)
                issue(2 * k + 3, buf_b, sem_b)
                return carry

            lax.fori_loop(0, NG // 2 - 1, pair_body, 0)
            wait(buf_a, sem_a)
            accum(NG - 2, buf_a)
            wait(buf_b, sem_b)
            accum(NG - 1, buf_b)

            pltpu.sync_copy(acc_v, out_hbm.at[t].at[pl.ds(base, BPW)])

    return pool


_pool = _make_pool_kernel()


def _mlp_body(p0, p1, p2, m, w1, b1, w2, b2, o):
    s = jnp.dot(p0[0], w1[0], preferred_element_type=jnp.float32)
    s = s + jnp.dot(p1[0], w1[1], preferred_element_type=jnp.float32)
    s = s + jnp.dot(p2[0], w1[2], preferred_element_type=jnp.float32)
    msum = jnp.sum(m[...], axis=1, keepdims=True)
    h = jnp.maximum(s / msum + b1[...], 0.0)
    o[...] = jnp.dot(h, w2[...], preferred_element_type=jnp.float32) + b2[...]


def _mlp(psum3, mask, W1r, b1r, W2, b2r):
    BB = 512
    grid = (B // BB,)
    return pl.pallas_call(
        _mlp_body,
        grid=grid,
        in_specs=[
            pl.BlockSpec((1, BB, EMB), lambda b: (0, b, 0)),
            pl.BlockSpec((1, BB, EMB), lambda b: (1, b, 0)),
            pl.BlockSpec((1, BB, EMB), lambda b: (2, b, 0)),
            pl.BlockSpec((BB, L), lambda b: (b, 0)),
            pl.BlockSpec((3, EMB, HID), lambda b: (0, 0, 0)),
            pl.BlockSpec((1, HID), lambda b: (0, 0)),
            pl.BlockSpec((HID, NCLS), lambda b: (0, 0)),
            pl.BlockSpec((1, NCLS), lambda b: (0, 0)),
        ],
        out_specs=pl.BlockSpec((BB, NCLS), lambda b: (b, 0)),
        out_shape=jax.ShapeDtypeStruct((B, NCLS), jnp.float32),
    )(psum3, psum3, psum3, mask, W1r, b1r, W2, b2r)


def kernel(shapes, colors, clusters, mask, shape_table, color_table,
           cluster_table, W1, b1, W2, b2):
    psum3 = _pool(shapes.reshape(-1), colors.reshape(-1),
                  clusters.reshape(-1),
                  shape_table, color_table, cluster_table)
    return _mlp(psum3, mask, W1.reshape(3, EMB, HID),
                b1.reshape(1, HID), W2, b2.reshape(1, NCLS))
